# baseline (speedup 1.0000x reference)
import functools

import jax
import jax.numpy as jnp
from jax import lax
from jax.experimental import pallas as pl
from jax.experimental.pallas import tpu as pltpu
from jax.experimental.pallas import tpu_sc as plsc

_N = 10000
_D = 128
_E = 320000
_LAYERS = 10

_NC = 2
_NS = 16
_NW = _NC * _NS
_EPW = _E // _NW
_CH = 128
_NCH = _EPW // _CH
_TAIL = _EPW - _NCH * _CH
_NP = 10240
_RPT = _NP // _NS
_RCH = 128
_NRCH = _RPT // _RCH


def _mesh():
    return plsc.VectorSubcoreMesh(core_axis_name="c", subcore_axis_name="s")


_NBUF = 2
_NSLOT = 2 * _NBUF


@functools.cache
def _sc_agg():
    def body(hw_hbm, src_hbm, dst_hbm, out_hbm, acc, *scr):
        rows = list(scr[:_NBUF])
        sidx = list(scr[_NBUF:_NBUF + _NSLOT])
        didx = list(scr[_NBUF + _NSLOT:_NBUF + 2 * _NSLOT])
        trows = scr[_NBUF + 2 * _NSLOT]
        tsidx = scr[_NBUF + 2 * _NSLOT + 1]
        tdidx = scr[_NBUF + 2 * _NSLOT + 2]
        base_sem = _NBUF + 2 * _NSLOT + 3
        gsem = list(scr[base_sem:base_sem + _NBUF])
        isem = list(scr[base_sem + _NBUF:])
        c = lax.axis_index("c")
        s = lax.axis_index("s")
        w = c * _NS + s

        z16 = jnp.zeros((16,), jnp.float32)

        def _fill(i, _):
            r = i // 8
            col = (i % 8) * 16
            rows[0][r, pl.ds(col, 16)] = z16
            return 0

        lax.fori_loop(0, _CH * 8, _fill, 0)

        base = w * _EPW

        def _fire_idx(i, slot):
            sl = pl.ds(base + i * _CH, _CH)
            pltpu.async_copy(src_hbm.at[sl], sidx[slot], isem[slot])
            pltpu.async_copy(dst_hbm.at[sl], didx[slot], isem[slot])

        def _wait_idx(i, slot):
            sl = pl.ds(base + i * _CH, _CH)
            pltpu.make_async_copy(src_hbm.at[sl], sidx[slot], isem[slot]).wait()
            pltpu.make_async_copy(dst_hbm.at[sl], didx[slot], isem[slot]).wait()

        def _fire_gather(i, b, slot):
            pltpu.async_copy(hw_hbm.at[sidx[slot]], rows[b], gsem[b])

        def _drain(i, b, slot):
            pltpu.make_async_copy(hw_hbm.at[sidx[slot]], rows[b],
                                  gsem[b]).wait()
            pltpu.sync_copy(rows[b], acc.at[didx[slot]], add=True)

        for i in range(_NSLOT):
            _fire_idx(i, i)
        for i in range(1, _NBUF):
            _wait_idx(i, i)
            _fire_gather(i, i, i)

        for k in range(_RPT // _CH):
            pltpu.sync_copy(rows[0], acc.at[pl.ds(s * _RPT + k * _CH, _CH)])
        _wait_idx(0, 0)
        _fire_gather(0, 0, 0)
        plsc.subcore_barrier()

        def _grp(kk, _):
            i0 = kk * _NSLOT
            for j in range(_NSLOT):
                i = i0 + j
                b = j % _NBUF
                _drain(i, b, j)

                @pl.when(i + _NSLOT < _NCH)
                def _():
                    _fire_idx(i + _NSLOT, j)

                s2 = (j + _NBUF) % _NSLOT
                _wait_idx(i + _NBUF, s2)
                _fire_gather(i + _NBUF, b, s2)
            return 0

        lax.fori_loop(0, _NCH // _NSLOT, _grp, 0)
        e0 = (_NCH // _NSLOT) * _NSLOT
        for j in range(_NCH % _NSLOT):
            _drain(e0 + j, j % _NBUF, j)
        tsl = pl.ds(base + _NCH * _CH, _TAIL)
        pltpu.sync_copy(src_hbm.at[tsl], tsidx)
        pltpu.sync_copy(dst_hbm.at[tsl], tdidx)
        pltpu.async_copy(hw_hbm.at[tsidx], trows, gsem[0]).wait()
        pltpu.sync_copy(trows, acc.at[tdidx], add=True)
        plsc.subcore_barrier()

        for k in range(_RPT // _CH):
            start = s * _RPT + k * _CH
            pltpu.sync_copy(acc.at[pl.ds(start, _CH)],
                            out_hbm.at[pl.ds(c * _NP + start, _CH)])

    return pl.kernel(
        body,
        out_type=jax.ShapeDtypeStruct((_NC * _NP, _D), jnp.float32),
        mesh=_mesh(),
        scratch_types=(
            [pltpu.VMEM_SHARED((_NP, _D), jnp.float32)]
            + [pltpu.VMEM((_CH, _D), jnp.float32) for _ in range(_NBUF)]
            + [pltpu.VMEM((_CH,), jnp.int32) for _ in range(2 * _NSLOT)]
            + [pltpu.VMEM((_TAIL, _D), jnp.float32),
               pltpu.VMEM((_TAIL,), jnp.int32),
               pltpu.VMEM((_TAIL,), jnp.int32)]
            + [pltpu.SemaphoreType.DMA for _ in range(_NBUF + _NSLOT)]
        ),
    )


_BR = 2000


def _tc_dinv(parts):
    def body(p_ref, o_ref):
        p = p_ref[...]
        deg = p[:_N, 0:1] + p[_NP:_NP + _N, 0:1] + 1.0
        o_ref[...] = lax.rsqrt(deg)

    return pl.pallas_call(
        body,
        out_shape=jax.ShapeDtypeStruct((_N, 1), jnp.float32),
    )(parts)


def _tc_first(x, W, dinv):
    def body(x_ref, w_ref, d_ref, o_ref):
        o_ref[...] = d_ref[...] * jnp.dot(
            x_ref[...], w_ref[...], preferred_element_type=jnp.float32
        )

    return pl.pallas_call(
        body,
        grid=(_N // _BR,),
        in_specs=[
            pl.BlockSpec((_BR, _D), lambda i: (i, 0)),
            pl.BlockSpec((_D, _D), lambda i: (0, 0)),
            pl.BlockSpec((_BR, 1), lambda i: (i, 0)),
        ],
        out_specs=pl.BlockSpec((_BR, _D), lambda i: (i, 0)),
        out_shape=jax.ShapeDtypeStruct((_N, _D), jnp.float32),
    )(x, W, dinv)


def _tc_layer(p0, p1, hw, dinv, b, g, bt, Wn):
    def body(p0_ref, p1_ref, hw_ref, d_ref, b_ref, g_ref, bt_ref, w_ref, o_ref):
        d = d_ref[...]
        t = d * (p0_ref[...] + p1_ref[...] + hw_ref[...]) + b_ref[...]
        mu = jnp.mean(t, axis=-1, keepdims=True)
        var = jnp.mean((t - mu) ** 2, axis=-1, keepdims=True)
        y = (t - mu) * lax.rsqrt(var + 1e-5) * g_ref[...] + bt_ref[...]
        y = y * jax.nn.sigmoid(y)
        o_ref[...] = d * jnp.dot(y, w_ref[...], preferred_element_type=jnp.float32)

    row = lambda i: (i, 0)
    fixed = lambda i: (0, 0)
    return pl.pallas_call(
        body,
        grid=(_N // _BR,),
        in_specs=[
            pl.BlockSpec((_BR, _D), row),
            pl.BlockSpec((_BR, _D), row),
            pl.BlockSpec((_BR, _D), row),
            pl.BlockSpec((_BR, 1), row),
            pl.BlockSpec((1, _D), fixed),
            pl.BlockSpec((1, _D), fixed),
            pl.BlockSpec((1, _D), fixed),
            pl.BlockSpec((_D, _D), fixed),
        ],
        out_specs=pl.BlockSpec((_BR, _D), row),
        out_shape=jax.ShapeDtypeStruct((_N, _D), jnp.float32),
    )(p0, p1, hw, dinv, b, g, bt, Wn)


def _tc_final(p0, p1, hw, dinv, b):
    def body(p0_ref, p1_ref, hw_ref, d_ref, b_ref, o_ref):
        o_ref[...] = d_ref[...] * (p0_ref[...] + p1_ref[...] + hw_ref[...]) + b_ref[...]

    row = lambda i: (i, 0)
    fixed = lambda i: (0, 0)
    return pl.pallas_call(
        body,
        grid=(_N // _BR,),
        in_specs=[
            pl.BlockSpec((_BR, _D), row),
            pl.BlockSpec((_BR, _D), row),
            pl.BlockSpec((_BR, _D), row),
            pl.BlockSpec((_BR, 1), row),
            pl.BlockSpec((1, _D), fixed),
        ],
        out_specs=pl.BlockSpec((_BR, _D), row),
        out_shape=jax.ShapeDtypeStruct((_N, _D), jnp.float32),
    )(p0, p1, hw, dinv, b)


def kernel(x, edge_index, Ws, bs, gammas, betas):
    src = edge_index[0]
    dst = edge_index[1]
    agg = _sc_agg()
    degp = agg(jnp.ones((_N, _D), jnp.float32), src, dst)
    dinv = _tc_dinv(degp)

    hw = _tc_first(x, Ws[0], dinv)
    for i in range(_LAYERS - 1):
        parts = agg(hw, src, dst)
        hw = _tc_layer(
            parts[:_N],
            parts[_NP:_NP + _N],
            hw,
            dinv,
            bs[i].reshape(1, _D),
            gammas[i].reshape(1, _D),
            betas[i].reshape(1, _D),
            Ws[i + 1],
        )
    parts = agg(hw, src, dst)
    return _tc_final(parts[:_N], parts[_NP:_NP + _N], hw, dinv, bs[_LAYERS - 1].reshape(1, _D))


# speedup vs baseline: 1.1466x; 1.1466x over previous
import functools

import jax
import jax.numpy as jnp
from jax import lax
from jax.experimental import pallas as pl
from jax.experimental.pallas import tpu as pltpu
from jax.experimental.pallas import tpu_sc as plsc

_N = 10000
_D = 128
_E = 320000
_LAYERS = 10

_NC = 2
_NS = 16
_NW = _NC * _NS
_EPW = _E // _NW
_CH = 80
_NCH = _EPW // _CH
_NP = 10240
_RPT = _NP // _NS
_RCH = 128
_NRCH = _RPT // _RCH


def _mesh():
    return plsc.VectorSubcoreMesh(core_axis_name="c", subcore_axis_name="s")


_NBUF = 4
_NSLOT = 2 * _NBUF


@functools.cache
def _sc_agg():
    def body(hw_hbm, src_hbm, dst_hbm, out_hbm, acc, *scr):
        rows = list(scr[:_NBUF])
        sidx = list(scr[_NBUF:_NBUF + _NSLOT])
        didx = list(scr[_NBUF + _NSLOT:_NBUF + 2 * _NSLOT])
        gsem = list(scr[_NBUF + 2 * _NSLOT:_NBUF + 2 * _NSLOT + _NBUF])
        isem = list(scr[_NBUF + 2 * _NSLOT + _NBUF:])
        c = lax.axis_index("c")
        s = lax.axis_index("s")
        w = c * _NS + s

        z16 = jnp.zeros((16,), jnp.float32)

        def _fill(i, _):
            r = i // 8
            col = (i % 8) * 16
            rows[0][r, pl.ds(col, 16)] = z16
            return 0

        lax.fori_loop(0, _CH * 8, _fill, 0)

        base = w * _EPW

        def _fire_idx(i, slot):
            sl = pl.ds(base + i * _CH, _CH)
            pltpu.async_copy(src_hbm.at[sl], sidx[slot], isem[slot])
            pltpu.async_copy(dst_hbm.at[sl], didx[slot], isem[slot])

        def _wait_idx(i, slot):
            sl = pl.ds(base + i * _CH, _CH)
            pltpu.make_async_copy(src_hbm.at[sl], sidx[slot], isem[slot]).wait()
            pltpu.make_async_copy(dst_hbm.at[sl], didx[slot], isem[slot]).wait()

        def _fire_gather(i, b, slot):
            pltpu.async_copy(hw_hbm.at[sidx[slot]], rows[b], gsem[b])

        def _drain(i, b, slot):
            pltpu.make_async_copy(hw_hbm.at[sidx[slot]], rows[b],
                                  gsem[b]).wait()
            pltpu.sync_copy(rows[b], acc.at[didx[slot]], add=True)

        for i in range(_NSLOT):
            _fire_idx(i, i)
        for i in range(1, _NBUF):
            _wait_idx(i, i)
            _fire_gather(i, i, i)

        for k in range(_RPT // _CH):
            pltpu.sync_copy(rows[0], acc.at[pl.ds(s * _RPT + k * _CH, _CH)])
        _wait_idx(0, 0)
        _fire_gather(0, 0, 0)
        plsc.subcore_barrier()

        def _grp(kk, _):
            i0 = kk * _NSLOT
            for j in range(_NSLOT):
                i = i0 + j
                b = j % _NBUF
                _drain(i, b, j)

                @pl.when(i + _NSLOT < _NCH)
                def _():
                    _fire_idx(i + _NSLOT, j)

                s2 = (j + _NBUF) % _NSLOT
                _wait_idx(i + _NBUF, s2)
                _fire_gather(i + _NBUF, b, s2)
            return 0

        lax.fori_loop(0, _NCH // _NSLOT, _grp, 0)
        e0 = (_NCH // _NSLOT) * _NSLOT
        _drain(e0, 0, 0)
        _wait_idx(_NCH - 1, (_NCH - 1) % _NSLOT)
        _fire_gather(_NCH - 1, (_NCH - 1) % _NBUF, (_NCH - 1) % _NSLOT)
        for j in range(1, _NCH % _NSLOT):
            _drain(e0 + j, j % _NBUF, j)
        plsc.subcore_barrier()

        for k in range(_RPT // _CH):
            start = s * _RPT + k * _CH
            pltpu.sync_copy(acc.at[pl.ds(start, _CH)],
                            out_hbm.at[pl.ds(c * _NP + start, _CH)])

    return pl.kernel(
        body,
        out_type=jax.ShapeDtypeStruct((_NC * _NP, _D), jnp.float32),
        mesh=_mesh(),
        scratch_types=(
            [pltpu.VMEM_SHARED((_NP, _D), jnp.float32)]
            + [pltpu.VMEM((_CH, _D), jnp.float32) for _ in range(_NBUF)]
            + [pltpu.VMEM((_CH,), jnp.int32) for _ in range(2 * _NSLOT)]
            + [pltpu.SemaphoreType.DMA for _ in range(_NBUF + _NSLOT)]
        ),
    )


_BR = 2000


def _tc_dinv(parts):
    def body(p_ref, o_ref):
        p = p_ref[...]
        deg = p[:_N, 0:1] + p[_NP:_NP + _N, 0:1] + 1.0
        o_ref[...] = lax.rsqrt(deg)

    return pl.pallas_call(
        body,
        out_shape=jax.ShapeDtypeStruct((_N, 1), jnp.float32),
    )(parts)


def _tc_first(x, W, dinv):
    def body(x_ref, w_ref, d_ref, o_ref):
        o_ref[...] = d_ref[...] * jnp.dot(
            x_ref[...], w_ref[...], preferred_element_type=jnp.float32
        )

    return pl.pallas_call(
        body,
        grid=(_N // _BR,),
        in_specs=[
            pl.BlockSpec((_BR, _D), lambda i: (i, 0)),
            pl.BlockSpec((_D, _D), lambda i: (0, 0)),
            pl.BlockSpec((_BR, 1), lambda i: (i, 0)),
        ],
        out_specs=pl.BlockSpec((_BR, _D), lambda i: (i, 0)),
        out_shape=jax.ShapeDtypeStruct((_N, _D), jnp.float32),
    )(x, W, dinv)


def _tc_layer(p0, p1, hw, dinv, b, g, bt, Wn):
    def body(p0_ref, p1_ref, hw_ref, d_ref, b_ref, g_ref, bt_ref, w_ref, o_ref):
        d = d_ref[...]
        t = d * (p0_ref[...] + p1_ref[...] + hw_ref[...]) + b_ref[...]
        mu = jnp.mean(t, axis=-1, keepdims=True)
        var = jnp.mean((t - mu) ** 2, axis=-1, keepdims=True)
        y = (t - mu) * lax.rsqrt(var + 1e-5) * g_ref[...] + bt_ref[...]
        y = y * jax.nn.sigmoid(y)
        o_ref[...] = d * jnp.dot(y, w_ref[...], preferred_element_type=jnp.float32)

    row = lambda i: (i, 0)
    fixed = lambda i: (0, 0)
    return pl.pallas_call(
        body,
        grid=(_N // _BR,),
        in_specs=[
            pl.BlockSpec((_BR, _D), row),
            pl.BlockSpec((_BR, _D), row),
            pl.BlockSpec((_BR, _D), row),
            pl.BlockSpec((_BR, 1), row),
            pl.BlockSpec((1, _D), fixed),
            pl.BlockSpec((1, _D), fixed),
            pl.BlockSpec((1, _D), fixed),
            pl.BlockSpec((_D, _D), fixed),
        ],
        out_specs=pl.BlockSpec((_BR, _D), row),
        out_shape=jax.ShapeDtypeStruct((_N, _D), jnp.float32),
    )(p0, p1, hw, dinv, b, g, bt, Wn)


def _tc_final(p0, p1, hw, dinv, b):
    def body(p0_ref, p1_ref, hw_ref, d_ref, b_ref, o_ref):
        o_ref[...] = d_ref[...] * (p0_ref[...] + p1_ref[...] + hw_ref[...]) + b_ref[...]

    row = lambda i: (i, 0)
    fixed = lambda i: (0, 0)
    return pl.pallas_call(
        body,
        grid=(_N // _BR,),
        in_specs=[
            pl.BlockSpec((_BR, _D), row),
            pl.BlockSpec((_BR, _D), row),
            pl.BlockSpec((_BR, _D), row),
            pl.BlockSpec((_BR, 1), row),
            pl.BlockSpec((1, _D), fixed),
        ],
        out_specs=pl.BlockSpec((_BR, _D), row),
        out_shape=jax.ShapeDtypeStruct((_N, _D), jnp.float32),
    )(p0, p1, hw, dinv, b)


def kernel(x, edge_index, Ws, bs, gammas, betas):
    src = edge_index[0]
    dst = edge_index[1]
    agg = _sc_agg()
    degp = agg(jnp.ones((_N, _D), jnp.float32), src, dst)
    dinv = _tc_dinv(degp)

    hw = _tc_first(x, Ws[0], dinv)
    for i in range(_LAYERS - 1):
        parts = agg(hw, src, dst)
        hw = _tc_layer(
            parts[:_N],
            parts[_NP:_NP + _N],
            hw,
            dinv,
            bs[i].reshape(1, _D),
            gammas[i].reshape(1, _D),
            betas[i].reshape(1, _D),
            Ws[i + 1],
        )
    parts = agg(hw, src, dst)
    return _tc_final(parts[:_N], parts[_NP:_NP + _N], hw, dinv, bs[_LAYERS - 1].reshape(1, _D))
